# Initial kernel scaffold; baseline (speedup 1.0000x reference)
import jax
import jax.numpy as jnp
from jax.experimental import pallas as pl


def kernel(x, gW0, gb0, gW1, gb1, SW0, Sb0, SW1, Sb1, SW2, Sb2, AW0, Ab0, AW1, Ab1, AW2, Ab2):
    raise NotImplementedError("write your pallas kernel here")


# speedup vs baseline: 1.5776x; 1.5776x over previous
import functools

import jax
import jax.numpy as jnp
import numpy as np
from jax.experimental import pallas as pl
from jax.experimental.pallas import tpu as pltpu

ENC = 8
SIG = 0.1


def _pos_enc(x_blk):
    t = jax.lax.broadcasted_iota(jnp.int32, (1, ENC // 2), 1).astype(jnp.float32) / ENC
    fr = (2.0 * np.pi) * jnp.exp(t * float(np.log(SIG)))
    x0 = x_blk[:, 0:1]
    x1 = x_blk[:, 1:2]
    return jnp.concatenate(
        [jnp.sin(x0 * fr), jnp.cos(x0 * fr), jnp.sin(x1 * fr), jnp.cos(x1 * fr)],
        axis=1,
    )


def _dense_body(x_ref, gW0_ref, gb0_ref, gW1_ref, gb1_ref,
                SW0_ref, Sb0_ref, SW1_ref, Sb1_ref, SW2_ref, Sb2_ref,
                AW0_ref, Ab0_ref, AW1_ref, Ab1_ref, AW2_ref, Ab2_ref,
                y_ref, *, n_experts):
    x_blk = x_ref[...]
    pe = _pos_enc(x_blk)

    h = jax.nn.relu(jnp.dot(pe, gW0_ref[...],
                            preferred_element_type=jnp.float32) + gb0_ref[...])
    logits = jnp.dot(h, gW1_ref[...],
                     preferred_element_type=jnp.float32) + gb1_ref[...]
    bm = logits.shape[0]
    idxs = jax.lax.broadcasted_iota(jnp.int32, (bm, n_experts), 1)
    m1 = jnp.max(logits, axis=1, keepdims=True)
    i1 = jnp.min(jnp.where(logits == m1, idxs, n_experts), axis=1, keepdims=True)
    masked = jnp.where(idxs == i1, -jnp.inf, logits)
    m2 = jnp.max(masked, axis=1, keepdims=True)
    i2 = jnp.min(jnp.where(masked == m2, idxs, n_experts), axis=1, keepdims=True)
    r = jnp.exp(m2 - m1)
    g1 = 1.0 / (1.0 + r)
    g2 = r / (1.0 + r)

    acc = jnp.zeros((bm, y_ref.shape[1]), dtype=jnp.float32)
    for e in range(n_experts):
        s = jax.nn.relu(jnp.dot(pe, SW0_ref[e],
                                preferred_element_type=jnp.float32) + Sb0_ref[e])
        s = jax.nn.relu(jnp.dot(s, SW1_ref[e],
                                preferred_element_type=jnp.float32) + Sb1_ref[e])
        iso = jnp.dot(s, SW2_ref[e],
                      preferred_element_type=jnp.float32) + Sb2_ref[e]
        a = jax.nn.relu(iso * AW0_ref[e][0][None, :] + Ab0_ref[e])
        a = jax.nn.relu(jnp.dot(a, AW1_ref[e],
                                preferred_element_type=jnp.float32) + Ab1_ref[e])
        o = jnp.dot(a, AW2_ref[e],
                    preferred_element_type=jnp.float32) + Ab2_ref[e]
        gate_e = g1 * (i1 == e) + g2 * (i2 == e)
        acc = acc + gate_e * o
    y_ref[...] = acc


def kernel(x, gW0, gb0, gW1, gb1, SW0, Sb0, SW1, Sb1, SW2, Sb2,
           AW0, Ab0, AW1, Ab1, AW2, Ab2):
    n = x.shape[0]
    n_experts = SW0.shape[0]
    g_out = AW2.shape[2]
    bm = min(256, n)
    grid = (n // bm,)

    full = lambda *shape: pl.BlockSpec(shape, lambda m: (0,) * len(shape))
    out = pl.pallas_call(
        functools.partial(_dense_body, n_experts=n_experts),
        grid=grid,
        in_specs=[
            pl.BlockSpec((bm, 2), lambda m: (m, 0)),
            full(*gW0.shape), full(*gb0.shape), full(*gW1.shape), full(*gb1.shape),
            full(*SW0.shape), full(*Sb0.shape), full(*SW1.shape), full(*Sb1.shape),
            full(*SW2.shape), full(*Sb2.shape),
            full(*AW0.shape), full(*Ab0.shape), full(*AW1.shape), full(*Ab1.shape),
            full(*AW2.shape), full(*Ab2.shape),
        ],
        out_specs=pl.BlockSpec((bm, g_out), lambda m: (m, 0)),
        out_shape=jax.ShapeDtypeStruct((n, g_out), jnp.float32),
        compiler_params=pltpu.CompilerParams(
            dimension_semantics=("arbitrary",),
        ),
    )(x, gW0, gb0, gW1, gb1, SW0, Sb0, SW1, Sb1, SW2, Sb2,
      AW0, Ab0, AW1, Ab1, AW2, Ab2)
    return out
